# baseline (speedup 1.0000x reference)
import functools

import jax
import jax.numpy as jnp
import numpy as np
from jax import lax
from jax.experimental import pallas as pl
from jax.experimental.pallas import tpu as pltpu
from jax.experimental.pallas import tpu_sc as plsc

BS, SEQ, EMB = 4, 4096, 2048
TOTAL = BS * EMB
NC, NS = 2, 16
NW = NC * NS
PER_W = TOTAL // NW
CHUNK = 128
NCH = PER_W // CHUNK

_ROT_A = (13, 15, 26, 6)
_ROT_B = (17, 29, 16, 24)


def _rotl_np(x, d):
    return ((x << np.uint32(d)) | (x >> np.uint32(32 - d))).astype(np.uint32)


def _threefry2x32_np(k1, k2, c1, c2):
    ks = (np.uint32(k1), np.uint32(k2),
          np.uint32(k1) ^ np.uint32(k2) ^ np.uint32(0x1BD11BDA))
    x0 = (np.asarray(c1, np.uint32) + ks[0]).astype(np.uint32)
    x1 = (np.asarray(c2, np.uint32) + ks[1]).astype(np.uint32)
    for i in range(5):
        for r in (_ROT_A if i % 2 == 0 else _ROT_B):
            x0 = (x0 + x1).astype(np.uint32)
            x1 = _rotl_np(x1, r) ^ x0
        x0 = (x0 + ks[(i + 1) % 3]).astype(np.uint32)
        x1 = (x1 + ks[(i + 2) % 3] + np.uint32(i + 1)).astype(np.uint32)
    return x0, x1


_SB1, _SB2 = _threefry2x32_np(np.uint32(0), np.uint32(42),
                              np.zeros(2, np.uint32),
                              np.arange(2, dtype=np.uint32))
_K1 = np.uint32(_SB1[1])
_K2 = np.uint32(_SB2[1])
_KS = (_K1, _K2, _K1 ^ _K2 ^ np.uint32(0x1BD11BDA))


def _i32(v):
    return int(np.int32(np.uint32(np.uint64(int(v)) & np.uint64(0xFFFFFFFF))))


def _gather_call(x2):
    mesh = plsc.VectorSubcoreMesh(core_axis_name="c", subcore_axis_name="s")

    @functools.partial(
        pl.kernel,
        mesh=mesh,
        out_type=jax.ShapeDtypeStruct((BS, EMB), jnp.float32),
        scratch_types=[
            pltpu.VMEM((NCH, CHUNK), jnp.int32),
            pltpu.VMEM((NCH, CHUNK, CHUNK), jnp.float32),
            pltpu.VMEM((PER_W,), jnp.float32),
            pltpu.SemaphoreType.DMA,
            pltpu.SemaphoreType.DMA,
        ],
        compiler_params=pltpu.CompilerParams(
            use_tc_tiling_on_sc=True, needs_layout_passes=False),
    )
    def body(x_hbm, out_hbm, row_v, slab_v, val_v, sem_g, sem_o):
        wid = lax.axis_index("s") * NC + lax.axis_index("c")
        base = wid * PER_W
        b = base // EMB
        j0 = base % EMB
        row_base = b * SEQ
        lane = lax.iota(jnp.int32, 16)

        gathers = []
        for c in range(NCH):
            for g in range(CHUNK // 16):
                x1 = base + c * CHUNK + g * 16 + lane + _i32(_KS[1])
                x0 = jnp.full((16,), _i32(_KS[0]), jnp.int32)
                for i in range(5):
                    for r in (_ROT_A if i % 2 == 0 else _ROT_B):
                        x0 = x0 + x1
                        x1 = ((x1 << r) |
                              lax.shift_right_logical(x1, 32 - r)) ^ x0
                    x0 = x0 + _i32(_KS[(i + 1) % 3])
                    x1 = x1 + _i32(int(_KS[(i + 2) % 3]) + i + 1)
                row_v[c, pl.ds(g * 16, 16)] = \
                    row_base + ((x0 ^ x1) & (SEQ - 1))
            gathers.append(pltpu.async_copy(
                x_hbm.at[row_v.at[c], pl.ds(j0 + c * CHUNK, CHUNK)],
                slab_v.at[c], sem_g))
        writes = []
        for c in range(NCH):
            gathers[c].wait()
            for k in range(CHUNK // 16):
                d = k * 16 + lane
                val_v[pl.ds(c * CHUNK + k * 16, 16)] = plsc.load_gather(
                    slab_v.at[c], [d, d])
            writes.append(pltpu.async_copy(
                val_v.at[pl.ds(c * CHUNK, CHUNK)],
                out_hbm.at[b, pl.ds(j0 + c * CHUNK, CHUNK)], sem_o))
        for w in writes:
            w.wait()

    return body(x2)


def kernel(output_tokens_from_bert):
    x = output_tokens_from_bert
    bs, seq, emb = x.shape
    return _gather_call(x.reshape(bs * seq, emb))


# speedup vs baseline: 1.0456x; 1.0456x over previous
import functools

import jax
import jax.numpy as jnp
import numpy as np
from jax import lax
from jax.experimental import pallas as pl
from jax.experimental.pallas import tpu as pltpu
from jax.experimental.pallas import tpu_sc as plsc

BS, SEQ, EMB = 4, 4096, 2048
TOTAL = BS * EMB
NC, NS = 2, 16
NW = NC * NS
PER_W = TOTAL // NW
CHUNK = 128
NCH = PER_W // CHUNK

_ROT_A = (13, 15, 26, 6)
_ROT_B = (17, 29, 16, 24)


def _rotl_np(x, d):
    return ((x << np.uint32(d)) | (x >> np.uint32(32 - d))).astype(np.uint32)


def _threefry2x32_np(k1, k2, c1, c2):
    ks = (np.uint32(k1), np.uint32(k2),
          np.uint32(k1) ^ np.uint32(k2) ^ np.uint32(0x1BD11BDA))
    x0 = (np.asarray(c1, np.uint32) + ks[0]).astype(np.uint32)
    x1 = (np.asarray(c2, np.uint32) + ks[1]).astype(np.uint32)
    for i in range(5):
        for r in (_ROT_A if i % 2 == 0 else _ROT_B):
            x0 = (x0 + x1).astype(np.uint32)
            x1 = _rotl_np(x1, r) ^ x0
        x0 = (x0 + ks[(i + 1) % 3]).astype(np.uint32)
        x1 = (x1 + ks[(i + 2) % 3] + np.uint32(i + 1)).astype(np.uint32)
    return x0, x1


_SB1, _SB2 = _threefry2x32_np(np.uint32(0), np.uint32(42),
                              np.zeros(2, np.uint32),
                              np.arange(2, dtype=np.uint32))
_K1 = np.uint32(_SB1[1])
_K2 = np.uint32(_SB2[1])
_KS = (_K1, _K2, _K1 ^ _K2 ^ np.uint32(0x1BD11BDA))


def _i32(v):
    return int(np.int32(np.uint32(np.uint64(int(v)) & np.uint64(0xFFFFFFFF))))


def _gather_call(x2):
    mesh = plsc.VectorSubcoreMesh(core_axis_name="c", subcore_axis_name="s")

    @functools.partial(
        pl.kernel,
        mesh=mesh,
        out_type=jax.ShapeDtypeStruct((BS, EMB), jnp.float32),
        scratch_types=[
            pltpu.VMEM((NCH, CHUNK), jnp.int32),
            pltpu.VMEM((NCH, CHUNK, CHUNK), jnp.float32),
            pltpu.VMEM((PER_W,), jnp.float32),
            pltpu.SemaphoreType.DMA,
            pltpu.SemaphoreType.DMA,
        ],
        compiler_params=pltpu.CompilerParams(
            use_tc_tiling_on_sc=True, needs_layout_passes=False),
    )
    def body(x_hbm, out_hbm, row_v, slab_v, val_v, sem_g, sem_o):
        wid = lax.axis_index("s") * NC + lax.axis_index("c")
        base = wid * PER_W
        b = base // EMB
        j0 = base % EMB
        row_base = b * SEQ
        lane = lax.iota(jnp.int32, 16)

        def sample_group(g, c):
            x1 = base + c * CHUNK + g * 16 + lane + _i32(_KS[1])
            x0 = jnp.full((16,), _i32(_KS[0]), jnp.int32)
            for i in range(5):
                for r in (_ROT_A if i % 2 == 0 else _ROT_B):
                    x0 = x0 + x1
                    x1 = ((x1 << r) |
                          lax.shift_right_logical(x1, 32 - r)) ^ x0
                x0 = x0 + _i32(_KS[(i + 1) % 3])
                x1 = x1 + _i32(int(_KS[(i + 2) % 3]) + i + 1)
            row_v[c, pl.ds(g * 16, 16)] = row_base + ((x0 ^ x1) & (SEQ - 1))
            return g + 1

        gathers = []
        for c in range(NCH):
            lax.fori_loop(0, CHUNK // 16, lambda g, _, c=c: (sample_group(g, c), _)[1], None)
            gathers.append(pltpu.async_copy(
                x_hbm.at[row_v.at[c], pl.ds(j0 + c * CHUNK, CHUNK)],
                slab_v.at[c], sem_g))
        writes = []
        for c in range(NCH):
            gathers[c].wait()

            def extract(k, _, c=c):
                d = k * 16 + lane
                val_v[pl.ds(c * CHUNK + k * 16, 16)] = plsc.load_gather(
                    slab_v.at[c], [d, d])
                return _

            lax.fori_loop(0, CHUNK // 16, extract, None)
            writes.append(pltpu.async_copy(
                val_v.at[pl.ds(c * CHUNK, CHUNK)],
                out_hbm.at[b, pl.ds(j0 + c * CHUNK, CHUNK)], sem_o))
        for w in writes:
            w.wait()

    return body(x2)


def kernel(output_tokens_from_bert):
    x = output_tokens_from_bert
    bs, seq, emb = x.shape
    return _gather_call(x.reshape(bs * seq, emb))
